# baseline (speedup 1.0000x reference)
import jax
import jax.numpy as jnp
from jax import lax
from jax.experimental import pallas as pl
from jax.experimental.pallas import tpu as pltpu
from jax.experimental.pallas import tpu_sc as plsc

NUM_CORES = 2
NUM_SUBCORES = 16
LANES = 16
NUM_WORKERS = NUM_CORES * NUM_SUBCORES
BATCH = 16384
BPW = BATCH // NUM_WORKERS
D = 32
NGROUPS = BPW // LANES
NSLOTS = 12

_DNUMS = lax.GatherDimensionNumbers(
    offset_dims=(), collapsed_slice_dims=(0,), start_index_map=(0,))


def _permute(v, idx):
    return lax.gather(v, idx[:, None], _DNUMS, (1,),
                      mode=lax.GatherScatterMode.PROMISE_IN_BOUNDS)


def _sc_body(user_hbm, item_hbm, qt_hbm, pt_hbm, bu_hbm, bi_hbm, out_hbm,
             uidx, iidx, qblk, pblk, bublk, biblk, out_v,
             semq, semp, sembu, sembi):
    cid = lax.axis_index("c")
    sid = lax.axis_index("s")
    wid = sid * NUM_CORES + cid
    base = wid * BPW

    pltpu.sync_copy(user_hbm.at[pl.ds(base, BPW)], uidx)
    pltpu.sync_copy(item_hbm.at[pl.ds(base, BPW)], iidx)

    lane = lax.iota(jnp.int32, LANES)
    klo = lane
    khi = lane + LANES
    zero16 = jnp.zeros((LANES,), jnp.int32)

    def fetch(s, uid, iid):
        uc = pl.multiple_of((uid >> 7) * 128, 128)
        ic = pl.multiple_of((iid >> 7) * 128, 128)
        pltpu.async_copy(qt_hbm.at[pl.ds(0, D), pl.ds(uc, 128)], qblk.at[s], semq)
        pltpu.async_copy(pt_hbm.at[pl.ds(0, D), pl.ds(ic, 128)], pblk.at[s], semp)
        pltpu.async_copy(bu_hbm.at[pl.ds(uc, 128)], bublk.at[s], sembu)
        pltpu.async_copy(bi_hbm.at[pl.ds(ic, 128)], biblk.at[s], sembi)

    def drain(s):
        pltpu.make_async_copy(qt_hbm.at[pl.ds(0, D), pl.ds(0, 128)], qblk.at[s], semq).wait()
        pltpu.make_async_copy(pt_hbm.at[pl.ds(0, D), pl.ds(0, 128)], pblk.at[s], semp).wait()
        pltpu.make_async_copy(bu_hbm.at[pl.ds(0, 128)], bublk.at[s], sembu).wait()
        pltpu.make_async_copy(bi_hbm.at[pl.ds(0, 128)], biblk.at[s], sembi).wait()

    def dot_one(s, uid, iid, j, res):
        ul = zero16 + (uid & 127)
        il = zero16 + (iid & 127)
        ss = zero16 + s
        q0 = plsc.load_gather(qblk, [ss, klo, ul])
        q1 = plsc.load_gather(qblk, [ss, khi, ul])
        p0 = plsc.load_gather(pblk, [ss, klo, il])
        p1 = plsc.load_gather(pblk, [ss, khi, il])
        v = q0 * p0 + q1 * p1
        for sh in (1, 2, 4, 8):
            v = v + _permute(v, lane ^ sh)
        bu = plsc.load_gather(bublk, [ss, ul])
        bi = plsc.load_gather(biblk, [ss, il])
        val = v + bu + bi
        return jnp.where(lane == j, val, res)

    uvec0 = uidx[pl.ds(0, LANES)]
    ivec0 = iidx[pl.ds(0, LANES)]
    for j in range(NSLOTS):
        fetch(j, uvec0[j], ivec0[j])

    def group_body(g, carry):
        gbase = pl.multiple_of(g * LANES, LANES)
        noff = pl.multiple_of(
            jnp.where(g + 1 < NGROUPS, (g + 1) * LANES, 0), LANES)
        uvec = uidx[pl.ds(gbase, LANES)]
        ivec = iidx[pl.ds(gbase, LANES)]
        unext = uidx[pl.ds(noff, LANES)]
        inext = iidx[pl.ds(noff, LANES)]
        res = jnp.zeros((LANES,), jnp.float32)
        for j in range(LANES):
            s = (g * LANES + j) % NSLOTS
            drain(s)
            res = dot_one(s, uvec[j], ivec[j], j, res)
            if j + NSLOTS < LANES:
                fetch(s, uvec[j + NSLOTS], ivec[j + NSLOTS])
            else:
                @pl.when(g + 1 < NGROUPS)
                def _():
                    fetch(s, unext[j + NSLOTS - LANES], inext[j + NSLOTS - LANES])
        out_v[pl.ds(gbase, LANES)] = res
        return carry

    lax.fori_loop(0, NGROUPS, group_body, 0)

    pltpu.sync_copy(out_v, out_hbm.at[pl.ds(base, BPW)])


_sc_call = pl.kernel(
    _sc_body,
    out_type=jax.ShapeDtypeStruct((BATCH,), jnp.float32),
    mesh=plsc.VectorSubcoreMesh(
        core_axis_name="c", subcore_axis_name="s",
        num_cores=NUM_CORES, num_subcores=NUM_SUBCORES),
    scratch_types=[
        pltpu.VMEM((BPW,), jnp.int32),
        pltpu.VMEM((BPW,), jnp.int32),
        pltpu.VMEM((NSLOTS, D, 128), jnp.float32),
        pltpu.VMEM((NSLOTS, D, 128), jnp.float32),
        pltpu.VMEM((NSLOTS, 128), jnp.float32),
        pltpu.VMEM((NSLOTS, 128), jnp.float32),
        pltpu.VMEM((BPW,), jnp.float32),
        pltpu.SemaphoreType.DMA,
        pltpu.SemaphoreType.DMA,
        pltpu.SemaphoreType.DMA,
        pltpu.SemaphoreType.DMA,
    ],
    compiler_params=pltpu.CompilerParams(needs_layout_passes=False),
)


@jax.jit
def kernel(user_id, item_id, Q, P, b_u, b_i):
    return _sc_call(
        user_id.astype(jnp.int32), item_id.astype(jnp.int32),
        Q.T, P.T, b_u.reshape(-1), b_i.reshape(-1))


# speedup vs baseline: 1.0088x; 1.0088x over previous
import jax
import jax.numpy as jnp
from jax import lax
from jax.experimental import pallas as pl
from jax.experimental.pallas import tpu as pltpu
from jax.experimental.pallas import tpu_sc as plsc

NUM_CORES = 2
NUM_SUBCORES = 16
LANES = 16
NUM_WORKERS = NUM_CORES * NUM_SUBCORES
BATCH = 16384
BPW = BATCH // NUM_WORKERS
D = 32
NGROUPS = BPW // LANES
NSLOTS = 8

_DNUMS = lax.GatherDimensionNumbers(
    offset_dims=(), collapsed_slice_dims=(0,), start_index_map=(0,))


def _permute(v, idx):
    return lax.gather(v, idx[:, None], _DNUMS, (1,),
                      mode=lax.GatherScatterMode.PROMISE_IN_BOUNDS)


def _sc_body(user_hbm, item_hbm, qt_hbm, pt_hbm, bu_hbm, bi_hbm, out_hbm,
             uidx, iidx, qblk, pblk, bublk, biblk, out_v,
             semq, semp, sembu, sembi):
    cid = lax.axis_index("c")
    sid = lax.axis_index("s")
    wid = sid * NUM_CORES + cid
    base = wid * BPW

    pltpu.sync_copy(user_hbm.at[pl.ds(base, BPW)], uidx)
    pltpu.sync_copy(item_hbm.at[pl.ds(base, BPW)], iidx)

    lane = lax.iota(jnp.int32, LANES)
    klo = lane
    khi = lane + LANES
    zero16 = jnp.zeros((LANES,), jnp.int32)

    def fetch(s, uid, iid):
        uc = pl.multiple_of((uid >> 7) * 128, 128)
        ic = pl.multiple_of((iid >> 7) * 128, 128)
        pltpu.async_copy(qt_hbm.at[pl.ds(0, D), pl.ds(uc, 128)], qblk.at[s], semq)
        pltpu.async_copy(pt_hbm.at[pl.ds(0, D), pl.ds(ic, 128)], pblk.at[s], semp)
        pltpu.async_copy(bu_hbm.at[pl.ds(uc, 128)], bublk.at[s], sembu)
        pltpu.async_copy(bi_hbm.at[pl.ds(ic, 128)], biblk.at[s], sembi)

    def drain(s):
        pltpu.make_async_copy(qt_hbm.at[pl.ds(0, D), pl.ds(0, 128)], qblk.at[s], semq).wait()
        pltpu.make_async_copy(pt_hbm.at[pl.ds(0, D), pl.ds(0, 128)], pblk.at[s], semp).wait()
        pltpu.make_async_copy(bu_hbm.at[pl.ds(0, 128)], bublk.at[s], sembu).wait()
        pltpu.make_async_copy(bi_hbm.at[pl.ds(0, 128)], biblk.at[s], sembi).wait()

    def dot_one(s, uid, iid, j, res):
        ul = zero16 + (uid & 127)
        il = zero16 + (iid & 127)
        ss = zero16 + s
        q0 = plsc.load_gather(qblk, [ss, klo, ul])
        q1 = plsc.load_gather(qblk, [ss, khi, ul])
        p0 = plsc.load_gather(pblk, [ss, klo, il])
        p1 = plsc.load_gather(pblk, [ss, khi, il])
        v = q0 * p0 + q1 * p1
        for sh in (1, 2, 4, 8):
            v = v + _permute(v, lane ^ sh)
        bu = plsc.load_gather(bublk, [ss, ul])
        bi = plsc.load_gather(biblk, [ss, il])
        val = v + bu + bi
        return jnp.where(lane == j, val, res)

    uvec0 = uidx[pl.ds(0, LANES)]
    ivec0 = iidx[pl.ds(0, LANES)]
    for j in range(NSLOTS):
        fetch(j, uvec0[j], ivec0[j])

    def group_body(g, carry):
        gbase = pl.multiple_of(g * LANES, LANES)
        noff = pl.multiple_of(
            jnp.where(g + 1 < NGROUPS, (g + 1) * LANES, 0), LANES)
        uvec = uidx[pl.ds(gbase, LANES)]
        ivec = iidx[pl.ds(gbase, LANES)]
        unext = uidx[pl.ds(noff, LANES)]
        inext = iidx[pl.ds(noff, LANES)]
        res = jnp.zeros((LANES,), jnp.float32)
        for j in range(LANES):
            s = j % NSLOTS
            drain(s)
            res = dot_one(s, uvec[j], ivec[j], j, res)
            if j + NSLOTS < LANES:
                fetch(s, uvec[j + NSLOTS], ivec[j + NSLOTS])
            else:
                @pl.when(g + 1 < NGROUPS)
                def _():
                    fetch(s, unext[j + NSLOTS - LANES], inext[j + NSLOTS - LANES])
        out_v[pl.ds(gbase, LANES)] = res
        return carry

    lax.fori_loop(0, NGROUPS, group_body, 0)

    pltpu.sync_copy(out_v, out_hbm.at[pl.ds(base, BPW)])


_sc_call = pl.kernel(
    _sc_body,
    out_type=jax.ShapeDtypeStruct((BATCH,), jnp.float32),
    mesh=plsc.VectorSubcoreMesh(
        core_axis_name="c", subcore_axis_name="s",
        num_cores=NUM_CORES, num_subcores=NUM_SUBCORES),
    scratch_types=[
        pltpu.VMEM((BPW,), jnp.int32),
        pltpu.VMEM((BPW,), jnp.int32),
        pltpu.VMEM((NSLOTS, D, 128), jnp.float32),
        pltpu.VMEM((NSLOTS, D, 128), jnp.float32),
        pltpu.VMEM((NSLOTS, 128), jnp.float32),
        pltpu.VMEM((NSLOTS, 128), jnp.float32),
        pltpu.VMEM((BPW,), jnp.float32),
        pltpu.SemaphoreType.DMA,
        pltpu.SemaphoreType.DMA,
        pltpu.SemaphoreType.DMA,
        pltpu.SemaphoreType.DMA,
    ],
    compiler_params=pltpu.CompilerParams(needs_layout_passes=False),
)


@jax.jit
def kernel(user_id, item_id, Q, P, b_u, b_i):
    return _sc_call(
        user_id.astype(jnp.int32), item_id.astype(jnp.int32),
        Q.T, P.T, b_u.reshape(-1), b_i.reshape(-1))
